# Initial kernel scaffold; baseline (speedup 1.0000x reference)
import functools

import jax
import jax.numpy as jnp
from jax import lax
from jax.experimental import pallas as pl
from jax.experimental.pallas import tpu as pltpu
from jax.experimental.pallas import tpu_sc as plsc

NUM_JOINTS = 55
FREE_JOINTS = (0, 12, 15, 16, 17, 22, 23, 24)

COS_COEF = (1.0, -0.5, 0.0416666679084301, -0.0013888889225199819,
            2.480158036632929e-05, -2.755716934643715e-07,
            2.0874666439851808e-09, -1.1452432752134811e-11,
            4.6812393913499284e-14, -1.2639207263246956e-16)
SINC_COEF = (1.0, -0.1666666716337204, 0.008333333767950535,
             -0.00019841270113829523, 2.7557316570892e-06,
             -2.5052036889405827e-08, 1.6058034069121874e-10,
             -7.638324651601525e-13, 2.76412049919259e-15,
             -6.789281746514762e-18)
VERS_COEF = (0.5, -0.0416666679084301, 0.0013888889225199819,
             -2.4801587642286904e-05, 2.7557317139326187e-07,
             -2.0876722572893414e-09, 1.1470280454617399e-11,
             -4.775413869678377e-14, 1.5401969333350387e-16,
             -3.45510317484285e-19)

NC = 2
NS = 16
L = 16
NW = NC * NS


def _horner(coefs, x):
    r = jnp.full((L,), coefs[-1], dtype=jnp.float32)
    for c in reversed(coefs[:-1]):
        r = r * x + jnp.float32(c)
    return r


def _make_sc_kernel(batch, pose_cols, out_cols, exp_dim, light_cols):
    b_per_w = batch // NW
    mesh = plsc.VectorSubcoreMesh(core_axis_name="c", subcore_axis_name="s")

    def body(ids_hbm, init_hbm, pose_hbm, cam_hbm, exp_hbm, light_hbm,
             out_pose_hbm, out_cam_hbm, out_exp_hbm, out_light_hbm,
             idx_v, pose_v, out_v, cam_v, exp_v, light_v,
             sem0, sem1, sem2, sem3):
        wid = lax.axis_index("s") * NC + lax.axis_index("c")
        base = wid * b_per_w

        pltpu.sync_copy(ids_hbm.at[pl.ds(base, b_per_w)], idx_v)
        cp_pose = pltpu.async_copy(pose_hbm.at[idx_v], pose_v, sem0)
        cp_cam = pltpu.async_copy(cam_hbm.at[idx_v], cam_v, sem1)
        cp_exp = pltpu.async_copy(exp_hbm.at[idx_v], exp_v, sem2)
        cp_light = pltpu.async_copy(light_hbm.at[idx_v], light_v, sem3)

        pltpu.sync_copy(init_hbm.at[pl.ds(base, b_per_w)], out_v)

        cp_pose.wait()
        for j in FREE_JOINTS:
            c3 = 3 * j
            c9 = 9 * j
            for h in range(b_per_w // L):
                b = lax.iota(jnp.int32, L) + jnp.int32(h * L)

                def col(c):
                    return jnp.full((L,), c, dtype=jnp.int32)

                ax = plsc.load_gather(pose_v, [b, col(c3)])
                ay = plsc.load_gather(pose_v, [b, col(c3 + 1)])
                az = plsc.load_gather(pose_v, [b, col(c3 + 2)])
                eps = jnp.float32(1e-8)
                axp, ayp, azp = ax + eps, ay + eps, az + eps
                x = axp * axp + ayp * ayp + azp * azp
                c = _horner(COS_COEF, x)
                s = _horner(SINC_COEF, x)
                v = _horner(VERS_COEF, x)
                vxy = v * ax * ay
                vxz = v * ax * az
                vyz = v * ay * az
                sx, sy, sz = s * ax, s * ay, s * az
                m = (c + v * ax * ax, vxy - sz, vxz + sy,
                     vxy + sz, c + v * ay * ay, vyz - sx,
                     vxz - sy, vyz + sx, c + v * az * az)
                for k in range(9):
                    plsc.store_scatter(out_v, [b, col(c9 + k)], m[k])

        cp_cam.wait()
        cp_exp.wait()
        cp_light.wait()

        row = pl.ds(base, b_per_w)
        pltpu.sync_copy(out_v, out_pose_hbm.at[row])
        pltpu.sync_copy(cam_v, out_cam_hbm.at[row])
        pltpu.sync_copy(exp_v, out_exp_hbm.at[row])
        pltpu.sync_copy(light_v, out_light_hbm.at[row])

    f32 = jnp.float32
    return pl.kernel(
        body,
        out_type=(
            jax.ShapeDtypeStruct((batch, out_cols), f32),
            jax.ShapeDtypeStruct((batch, 3), f32),
            jax.ShapeDtypeStruct((batch, exp_dim), f32),
            jax.ShapeDtypeStruct((batch, light_cols), f32),
        ),
        mesh=mesh,
        scratch_types=[
            pltpu.VMEM((b_per_w,), jnp.int32),
            pltpu.VMEM((b_per_w, pose_cols), f32),
            pltpu.VMEM((b_per_w, out_cols), f32),
            pltpu.VMEM((b_per_w, 3), f32),
            pltpu.VMEM((b_per_w, exp_dim), f32),
            pltpu.VMEM((b_per_w, light_cols), f32),
            pltpu.SemaphoreType.DMA,
            pltpu.SemaphoreType.DMA,
            pltpu.SemaphoreType.DMA,
            pltpu.SemaphoreType.DMA,
        ],
    )


@jax.jit
def kernel(frame_ids, init_full_pose, init_cam, pose_table, cam_table,
           exp_table, light_table):
    del init_cam
    batch = frame_ids.shape[0]
    num_frames, nj, _ = pose_table.shape
    exp_dim = exp_table.shape[1]
    nl = light_table.shape[1]

    ids = frame_ids.astype(jnp.int32)
    init_flat = init_full_pose.reshape(batch, nj * 9)
    pose_flat = pose_table.reshape(num_frames, nj * 3)
    light_flat = light_table.reshape(num_frames, nl * 3)

    sc = _make_sc_kernel(batch, nj * 3, nj * 9, exp_dim, nl * 3)
    full_pose, cam, exp, light = sc(ids, init_flat, pose_flat, cam_table,
                                    exp_table, light_flat)
    return (full_pose.reshape(batch, nj, 3, 3), cam, exp,
            light.reshape(batch, nl, 3))


# speedup vs baseline: 1.7786x; 1.7786x over previous
import jax
import jax.numpy as jnp
from jax import lax
from jax.experimental import pallas as pl
from jax.experimental.pallas import tpu as pltpu
from jax.experimental.pallas import tpu_sc as plsc

FREE_JOINTS = (0, 12, 15, 16, 17, 22, 23, 24)

COS_COEF = (1.0, -0.5, 0.0416666679084301, -0.0013888889225199819,
            2.480158036632929e-05, -2.755716934643715e-07,
            2.0874666439851808e-09, -1.1452432752134811e-11,
            4.6812393913499284e-14, -1.2639207263246956e-16)
SINC_COEF = (1.0, -0.1666666716337204, 0.008333333767950535,
             -0.00019841270113829523, 2.7557316570892e-06,
             -2.5052036889405827e-08, 1.6058034069121874e-10,
             -7.638324651601525e-13, 2.76412049919259e-15,
             -6.789281746514762e-18)
VERS_COEF = (0.5, -0.0416666679084301, 0.0013888889225199819,
             -2.4801587642286904e-05, 2.7557317139326187e-07,
             -2.0876722572893414e-09, 1.1470280454617399e-11,
             -4.775413869678377e-14, 1.5401969333350387e-16,
             -3.45510317484285e-19)

NC = 2
NS = 16
L = 16
NW = NC * NS


def _horner(coefs, x):
    r = jnp.full((L,), coefs[-1], dtype=jnp.float32)
    for c in reversed(coefs[:-1]):
        r = r * x + jnp.float32(c)
    return r


def _pad8(n):
    return ((n + 7) // 8) * 8


def _make_sc_kernel(batch, nj, exp_dim, light_cols):
    b_per_w = batch // NW
    out_cols = nj * 9
    mesh = plsc.VectorSubcoreMesh(core_axis_name="c", subcore_axis_name="s")
    exp_p = _pad8(exp_dim)
    light_p = _pad8(light_cols)

    def body(ids_hbm, init_hbm, pose_hbm, cam_hbm, exp_hbm, light_hbm,
             out_pose_hbm, out_cam_hbm, out_exp_hbm, out_light_hbm,
             idx_v, pose_v, cam_v, exp_v, light_v, out_v,
             sem0, sem1, sem2, sem3):
        wid = lax.axis_index("s") * NC + lax.axis_index("c")
        base = wid * b_per_w

        pltpu.sync_copy(ids_hbm.at[pl.ds(base, b_per_w)], idx_v)
        cp_pose = pltpu.async_copy(pose_hbm.at[idx_v], pose_v, sem0)
        cp_cam = pltpu.async_copy(cam_hbm.at[idx_v], cam_v, sem1)
        cp_exp = pltpu.async_copy(exp_hbm.at[idx_v], exp_v, sem2)
        cp_light = pltpu.async_copy(light_hbm.at[idx_v], light_v, sem3)

        pltpu.sync_copy(init_hbm.at[pl.ds(base, b_per_w)], out_v)

        cp_pose.wait()
        for r, j in enumerate(FREE_JOINTS):
            c3 = 3 * r
            c9 = 9 * j
            for h in range(b_per_w // L):
                b = lax.iota(jnp.int32, L) + jnp.int32(h * L)

                def col(c):
                    return jnp.full((L,), c, dtype=jnp.int32)

                ax = plsc.load_gather(pose_v, [b, col(c3)])
                ay = plsc.load_gather(pose_v, [b, col(c3 + 1)])
                az = plsc.load_gather(pose_v, [b, col(c3 + 2)])
                eps = jnp.float32(1e-8)
                axp, ayp, azp = ax + eps, ay + eps, az + eps
                x = axp * axp + ayp * ayp + azp * azp
                c = _horner(COS_COEF, x)
                s = _horner(SINC_COEF, x)
                v = _horner(VERS_COEF, x)
                vxy = v * ax * ay
                vxz = v * ax * az
                vyz = v * ay * az
                sx, sy, sz = s * ax, s * ay, s * az
                m = (c + v * ax * ax, vxy - sz, vxz + sy,
                     vxy + sz, c + v * ay * ay, vyz - sx,
                     vxz - sy, vyz + sx, c + v * az * az)
                for k in range(9):
                    plsc.store_scatter(out_v, [b, col(c9 + k)], m[k])

        row = pl.ds(base, b_per_w)
        pltpu.sync_copy(out_v, out_pose_hbm.at[row])
        cp_cam.wait()
        pltpu.sync_copy(cam_v.at[:, pl.ds(0, 3)], out_cam_hbm.at[row])

        def split_copy(src_ref, dst_ref, width):
            main = (width // 8) * 8
            pltpu.sync_copy(src_ref.at[:, pl.ds(0, main)],
                            dst_ref.at[row, pl.ds(0, main)])
            if width > main:
                pltpu.sync_copy(src_ref.at[:, pl.ds(main, width - main)],
                                dst_ref.at[row, pl.ds(main, width - main)])

        cp_exp.wait()
        split_copy(exp_v, out_exp_hbm, exp_dim)
        cp_light.wait()
        split_copy(light_v, out_light_hbm, light_cols)

    f32 = jnp.float32
    return pl.kernel(
        body,
        out_type=(
            jax.ShapeDtypeStruct((batch, out_cols), f32),
            jax.ShapeDtypeStruct((batch, 3), f32),
            jax.ShapeDtypeStruct((batch, exp_dim), f32),
            jax.ShapeDtypeStruct((batch, light_cols), f32),
        ),
        mesh=mesh,
        compiler_params=pltpu.CompilerParams(
            needs_layout_passes=False, use_tc_tiling_on_sc=False),
        scratch_types=[
            pltpu.VMEM((b_per_w,), jnp.int32),
            pltpu.VMEM((b_per_w, 24), f32),
            pltpu.VMEM((b_per_w, 8), f32),
            pltpu.VMEM((b_per_w, exp_p), f32),
            pltpu.VMEM((b_per_w, light_p), f32),
            pltpu.VMEM((b_per_w, out_cols), f32),
            pltpu.SemaphoreType.DMA,
            pltpu.SemaphoreType.DMA,
            pltpu.SemaphoreType.DMA,
            pltpu.SemaphoreType.DMA,
        ],
    )


@jax.jit
def kernel(frame_ids, init_full_pose, init_cam, pose_table, cam_table,
           exp_table, light_table):
    del init_cam
    batch = frame_ids.shape[0]
    num_frames, nj, _ = pose_table.shape
    exp_dim = exp_table.shape[1]
    nl = light_table.shape[1]

    ids = frame_ids.astype(jnp.int32)
    init_flat = init_full_pose.reshape(batch, nj * 9)
    free = jnp.asarray(FREE_JOINTS, dtype=jnp.int32)

    def pad8(a):
        n = a.shape[1]
        p = _pad8(n)
        if p == n:
            return a
        return jnp.pad(a, ((0, 0), (0, p - n)))

    pose24 = pose_table[:, free, :].reshape(num_frames, 24)
    cam_p = pad8(cam_table)
    exp_p = pad8(exp_table)
    light_p = pad8(light_table.reshape(num_frames, nl * 3))

    sc = _make_sc_kernel(batch, nj, exp_dim, nl * 3)
    full_pose, cam, exp, light = sc(ids, init_flat, pose24, cam_p,
                                    exp_p, light_p)
    return (full_pose.reshape(batch, nj, 3, 3), cam, exp,
            light.reshape(batch, nl, 3))
